# baseline (speedup 1.0000x reference)
import functools

import jax
import jax.numpy as jnp
from jax import lax
from jax.experimental import pallas as pl
from jax.experimental.pallas import tpu as pltpu
from jax.experimental.pallas import tpu_sc as plsc

ROWS, COLS = 4096, 50
D = 768
V = 64
NC, NS = 2, 16
NW = NC * NS
R_PER_W = ROWS // NW

_mesh = plsc.VectorSubcoreMesh(core_axis_name="c", subcore_axis_name="s")


@functools.partial(
    pl.kernel,
    mesh=_mesh,
    out_type=jax.ShapeDtypeStruct((ROWS, COLS, D), jnp.float32),
    scratch_types=[
        pltpu.VMEM((V, D), jnp.float32),
        pltpu.VMEM((R_PER_W, COLS), jnp.int32),
        pltpu.VMEM((COLS, D), jnp.float32),
        pltpu.SemaphoreType.DMA,
    ],
)
def _embed(table_hbm, idx_hbm, out_hbm, table_v, idx_v, buf, sem):
    cid = lax.axis_index("c")
    sid = lax.axis_index("s")
    wid = sid * NC + cid
    base = wid * R_PER_W

    pltpu.sync_copy(table_hbm, table_v)
    pltpu.sync_copy(idx_hbm.at[pl.ds(base, R_PER_W)], idx_v)

    def body(n, carry):
        for j in range(COLS):
            g0 = min((j // 16) * 16, COLS - 16)
            t = idx_v[n, pl.ds(g0, 16)][j - g0]
            for c in range(D // 16):
                buf[j, pl.ds(c * 16, 16)] = table_v[t, pl.ds(c * 16, 16)]
        pltpu.sync_copy(buf, out_hbm.at[base + n])
        return carry

    lax.fori_loop(0, R_PER_W, body, 0)


def kernel(indices, table):
    return _embed(table, indices.astype(jnp.int32))


# speedup vs baseline: 2.9446x; 2.9446x over previous
import functools

import jax
import jax.numpy as jnp
from jax import lax
from jax.experimental import pallas as pl
from jax.experimental.pallas import tpu as pltpu
from jax.experimental.pallas import tpu_sc as plsc

ROWS, COLS = 4096, 50
D = 768
V = 64
NC, NS = 2, 16
NW = NC * NS
R_PER_W = ROWS // NW
NBUF = 2

_mesh = plsc.VectorSubcoreMesh(core_axis_name="c", subcore_axis_name="s")


@functools.partial(
    pl.kernel,
    mesh=_mesh,
    out_type=jax.ShapeDtypeStruct((ROWS, COLS, D), jnp.float32),
    scratch_types=[
        pltpu.VMEM_SHARED((V, D), jnp.float32),
        pltpu.VMEM((R_PER_W, COLS), jnp.int32),
        pltpu.VMEM((NBUF, COLS, D), jnp.float32),
        pltpu.SemaphoreType.DMA(NBUF),
        pltpu.SemaphoreType.DMA(NBUF),
    ],
)
def _embed(table_hbm, idx_hbm, out_hbm, table_sp, idx_v, ring, sem_l, sem_s):
    cid = lax.axis_index("c")
    sid = lax.axis_index("s")
    wid = sid * NC + cid
    base = wid * R_PER_W

    @pl.when(sid == 0)
    def _():
        pltpu.sync_copy(table_hbm, table_sp)

    plsc.subcore_barrier()

    pltpu.sync_copy(idx_hbm.at[pl.ds(base, R_PER_W)], idx_v)

    def fill(n, b):
        vecs = [idx_v[n, pl.ds(g, 16)] for g in (0, 16, 32, 34)]
        for j in range(COLS):
            t = vecs[j // 16][j % 16] if j < 48 else vecs[3][j - 34]
            pltpu.async_copy(table_sp.at[t], ring.at[b, j], sem_l.at[b])

    def fill_wait(n, b):
        for j in range(COLS):
            pltpu.make_async_copy(table_sp.at[0], ring.at[b, j],
                                  sem_l.at[b]).wait()

    def store(n, b):
        pltpu.async_copy(ring.at[b], out_hbm.at[base + n], sem_s.at[b])

    def store_wait(n, b):
        pltpu.make_async_copy(ring.at[b], out_hbm.at[base + n],
                              sem_s.at[b]).wait()

    def step(n, b, wait_old_store):
        if wait_old_store:
            store_wait(n - 2, b)
        fill(n, b)
        fill_wait(n, b)
        store(n, b)

    step(0, 0, wait_old_store=False)
    step(1, 1, wait_old_store=False)

    def body(q, carry):
        n = 2 + q * 2
        step(n, 0, wait_old_store=True)
        step(n + 1, 1, wait_old_store=True)
        return carry

    lax.fori_loop(0, R_PER_W // 2 - 1, body, 0)

    store_wait(R_PER_W - 2, 0)
    store_wait(R_PER_W - 1, 1)


def kernel(indices, table):
    return _embed(table, indices.astype(jnp.int32))


# speedup vs baseline: 2.9452x; 1.0002x over previous
import functools

import jax
import jax.numpy as jnp
from jax import lax
from jax.experimental import pallas as pl
from jax.experimental.pallas import tpu as pltpu
from jax.experimental.pallas import tpu_sc as plsc

ROWS, COLS = 4096, 50
D = 768
V = 64
NC, NS = 2, 16
NW = NC * NS
R_PER_W = ROWS // NW
NBUF = 2

_mesh = plsc.VectorSubcoreMesh(core_axis_name="c", subcore_axis_name="s")


@functools.partial(
    pl.kernel,
    mesh=_mesh,
    compiler_params=pltpu.CompilerParams(use_tc_tiling_on_sc=True),
    out_type=jax.ShapeDtypeStruct((ROWS, COLS, D), jnp.float32),
    scratch_types=[
        pltpu.VMEM_SHARED((V, D), jnp.float32),
        pltpu.VMEM((R_PER_W, COLS), jnp.int32),
        pltpu.VMEM((NBUF, COLS, D), jnp.float32),
        pltpu.SemaphoreType.DMA(NBUF),
        pltpu.SemaphoreType.DMA(NBUF),
    ],
)
def _embed(table_hbm, idx_hbm, out_hbm, table_sp, idx_v, ring, sem_l, sem_s):
    cid = lax.axis_index("c")
    sid = lax.axis_index("s")
    wid = sid * NC + cid
    base = wid * R_PER_W

    @pl.when(sid == 0)
    def _():
        pltpu.sync_copy(table_hbm, table_sp)

    plsc.subcore_barrier()

    pltpu.sync_copy(idx_hbm.at[pl.ds(base, R_PER_W)], idx_v)

    def fill(n, b):
        vecs = [idx_v[n, pl.ds(g, 16)] for g in (0, 16, 32, 34)]
        for j in range(COLS):
            t = vecs[j // 16][j % 16] if j < 48 else vecs[3][j - 34]
            pltpu.async_copy(table_sp.at[t], ring.at[b, j], sem_l.at[b])

    def fill_wait(n, b):
        for j in range(COLS):
            pltpu.make_async_copy(table_sp.at[0], ring.at[b, j],
                                  sem_l.at[b]).wait()

    def store(n, b):
        pltpu.async_copy(ring.at[b], out_hbm.at[base + n], sem_s.at[b])

    def store_wait(n, b):
        pltpu.make_async_copy(ring.at[b], out_hbm.at[base + n],
                              sem_s.at[b]).wait()

    def step(n, b, wait_old_store):
        if wait_old_store:
            store_wait(n - 2, b)
        fill(n, b)
        fill_wait(n, b)
        store(n, b)

    step(0, 0, wait_old_store=False)
    step(1, 1, wait_old_store=False)

    def body(q, carry):
        n = 2 + q * 2
        step(n, 0, wait_old_store=True)
        step(n + 1, 1, wait_old_store=True)
        return carry

    lax.fori_loop(0, R_PER_W // 2 - 1, body, 0)

    store_wait(R_PER_W - 2, 0)
    store_wait(R_PER_W - 1, 1)


def kernel(indices, table):
    return _embed(table, indices.astype(jnp.int32))


# speedup vs baseline: 7.1239x; 2.4188x over previous
import functools

import jax
import jax.numpy as jnp
from jax import lax
from jax.experimental import pallas as pl
from jax.experimental.pallas import tpu as pltpu
from jax.experimental.pallas import tpu_sc as plsc

ROWS, COLS = 4096, 50
D = 768
V = 64
NC, NS = 2, 16
NW = NC * NS
I_PER_W = ROWS // NW
HALF = I_PER_W // 2
NBUF = 2

_mesh = plsc.VectorSubcoreMesh(core_axis_name="c", subcore_axis_name="s")


@functools.partial(
    pl.kernel,
    mesh=_mesh,
    out_type=jax.ShapeDtypeStruct((COLS, ROWS, D), jnp.float32),
    scratch_types=[
        pltpu.VMEM_SHARED((V, D), jnp.float32),
        pltpu.VMEM((COLS, I_PER_W), jnp.int32),
        pltpu.VMEM((NBUF, HALF, D), jnp.float32),
        pltpu.SemaphoreType.DMA(NBUF),
        pltpu.SemaphoreType.DMA(NBUF),
    ],
)
def _embed(table_hbm, idxt_hbm, out_hbm, table_sp, idx_v, ring, sem_l, sem_s):
    cid = lax.axis_index("c")
    sid = lax.axis_index("s")
    wid = sid * NC + cid
    base = wid * I_PER_W

    @pl.when(sid == 0)
    def _():
        pltpu.sync_copy(table_hbm, table_sp)

    plsc.subcore_barrier()

    pltpu.sync_copy(idxt_hbm.at[:, pl.ds(base, I_PER_W)], idx_v)

    def fill(j, h):
        vecs = [idx_v[j, pl.ds(h * HALF + g, 16)] for g in (0, 16, 32, 48)]
        for k in range(HALF):
            t = vecs[k // 16][k % 16]
            pltpu.async_copy(table_sp.at[t], ring.at[h, k], sem_l.at[h])

    def fill_wait(h):
        for k in range(HALF):
            pltpu.make_async_copy(table_sp.at[0], ring.at[h, k],
                                  sem_l.at[h]).wait()

    def store(j, h):
        pltpu.async_copy(ring.at[h], out_hbm.at[j, pl.ds(base + h * HALF,
                                                         HALF)], sem_s.at[h])

    def store_wait(j, h):
        pltpu.make_async_copy(ring.at[h], out_hbm.at[j, pl.ds(base + h * HALF,
                                                              HALF)],
                              sem_s.at[h]).wait()

    def step(j, h, wait_old_store):
        if wait_old_store:
            store_wait(j - 1, h)
        fill(j, h)
        fill_wait(h)
        store(j, h)

    step(0, 0, wait_old_store=False)
    step(0, 1, wait_old_store=False)

    def body(j, carry):
        step(j, 0, wait_old_store=True)
        step(j, 1, wait_old_store=True)
        return carry

    lax.fori_loop(1, COLS, body, 0)

    store_wait(COLS - 1, 0)
    store_wait(COLS - 1, 1)


def kernel(indices, table):
    out = _embed(table, indices.astype(jnp.int32).T)
    return jnp.transpose(out, (1, 0, 2))


# speedup vs baseline: 7.1414x; 1.0025x over previous
import functools

import jax
import jax.numpy as jnp
from jax import lax
from jax.experimental import pallas as pl
from jax.experimental.pallas import tpu as pltpu
from jax.experimental.pallas import tpu_sc as plsc

ROWS, COLS = 4096, 50
D = 768
V = 64
NC, NS = 2, 16
NW = NC * NS
I_PER_W = ROWS // NW
HALF = I_PER_W // 2
NBUF = 2

_mesh = plsc.VectorSubcoreMesh(core_axis_name="c", subcore_axis_name="s")


@functools.partial(
    pl.kernel,
    mesh=_mesh,
    out_type=jax.ShapeDtypeStruct((COLS, ROWS, D), jnp.float32),
    scratch_types=[
        pltpu.VMEM_SHARED((V, D), jnp.float32),
        pltpu.VMEM((COLS, I_PER_W), jnp.int32),
        pltpu.VMEM((NBUF, HALF, D), jnp.float32),
        pltpu.SemaphoreType.DMA(NBUF),
        pltpu.SemaphoreType.DMA(NBUF),
    ],
)
def _embed(table_hbm, idxt_hbm, out_hbm, table_sp, idx_v, ring, sem_l, sem_s):
    cid = lax.axis_index("c")
    sid = lax.axis_index("s")
    wid = sid * NC + cid
    base = wid * I_PER_W

    @pl.when(sid == 0)
    def _():
        pltpu.sync_copy(table_hbm, table_sp)

    plsc.subcore_barrier()

    pltpu.sync_copy(idxt_hbm.at[:, pl.ds(base, I_PER_W)], idx_v)

    def fill(j, h):
        vecs = [idx_v[j, pl.ds(h * HALF + g, 16)] for g in (0, 16, 32, 48)]
        for k in range(HALF):
            t = vecs[k // 16][k % 16]
            pltpu.async_copy(table_sp.at[t], ring.at[h, k], sem_l.at[h])

    def fill_wait(h):
        pltpu.make_async_copy(out_hbm.at[0, pl.ds(0, HALF)], ring.at[h],
                              sem_l.at[h]).wait()

    def store(j, h):
        pltpu.async_copy(ring.at[h], out_hbm.at[j, pl.ds(base + h * HALF,
                                                         HALF)], sem_s.at[h])

    def store_wait(j, h):
        pltpu.make_async_copy(ring.at[h], out_hbm.at[j, pl.ds(base + h * HALF,
                                                              HALF)],
                              sem_s.at[h]).wait()

    def step(j, h, wait_old_store):
        if wait_old_store:
            store_wait(j - 1, h)
        fill(j, h)
        fill_wait(h)
        store(j, h)

    step(0, 0, wait_old_store=False)
    step(0, 1, wait_old_store=False)

    def body(j, carry):
        step(j, 0, wait_old_store=True)
        step(j, 1, wait_old_store=True)
        return carry

    lax.fori_loop(1, COLS, body, 0)

    store_wait(COLS - 1, 0)
    store_wait(COLS - 1, 1)


def kernel(indices, table):
    out = _embed(table, indices.astype(jnp.int32).T)
    return jnp.transpose(out, (1, 0, 2))
